# baseline (speedup 1.0000x reference)
import jax
import jax.numpy as jnp
from jax import lax
from jax.experimental import pallas as pl
from jax.experimental.pallas import tpu as pltpu
from jax.experimental.pallas import tpu_sc as plsc

N_NODES = 10000
N_EDGES = 320000
D_FEAT = 128
DH = D_FEAT // 2

NC = 2
NS = 16
NW = NC * NS

NP = 10240
ROWS_W = NP // NW
ROWS_T = NP // NS

E_TILE = N_EDGES // NS
CH = 128
KCH = N_EDGES // CH
K_W = KCH // NW
NEXTRA = KCH - NW * K_W
NBUF = 4

_mesh = plsc.VectorSubcoreMesh(core_axis_name="c", subcore_axis_name="s")
_params = pltpu.CompilerParams(needs_layout_passes=False,
                               use_tc_tiling_on_sc=False)


def _rsqrt_newton(v):
  i = plsc.bitcast(v, jnp.int32)
  y = plsc.bitcast(jnp.full((16,), 0x5F3759DF, jnp.int32)
                   - lax.shift_right_logical(i, 1), jnp.float32)
  half = v * 0.5
  for _ in range(3):
    y = y * (1.5 - half * y * y)
  return jnp.where(v > 0.5, y, jnp.zeros((16,), jnp.float32))


def _deg_scale_body(row_hbm, xp_hbm, xs0_hbm, xs1_hbm, dis_hbm,
                    row_v, deg_v, acc_v, red_v, x0_v, x1_v,
                    deg_all):
  c = lax.axis_index("c")
  t = lax.axis_index("s")

  def _z(i, _):
    deg_v[pl.ds(i * 16, 16)] = jnp.zeros((16,), jnp.float32)
    return 0
  lax.fori_loop(0, NP // 16, _z, 0)

  pltpu.sync_copy(row_hbm.at[pl.ds(t * E_TILE, E_TILE)], row_v)
  ones = jnp.ones((16,), jnp.float32)

  def _hist(i, _):
    for u in range(5):
      idx = row_v[pl.ds((i * 5 + u) * 16, 16)]
      plsc.addupdate_scatter(deg_v, [idx], ones)
    return 0
  lax.fori_loop(0, E_TILE // 16 // 5, _hist, 0)

  pltpu.sync_copy(deg_v, deg_all.at[t])
  plsc.subcore_barrier()

  pltpu.sync_copy(deg_all.at[pl.ds(0, NS), pl.ds(t * ROWS_T, ROWS_T)], red_v)

  def _add(i, _):
    o = i * 16
    v = red_v[0, pl.ds(o, 16)]
    for j in range(1, NS):
      v = v + red_v[j, pl.ds(o, 16)]
    acc_v[pl.ds(o, 16)] = v
    return 0
  lax.fori_loop(0, ROWS_T // 16, _add, 0)

  def _newton(i, _):
    acc_v[pl.ds(i * 16, 16)] = _rsqrt_newton(acc_v[pl.ds(i * 16, 16)])
    return 0
  lax.fori_loop(0, ROWS_T // 16, _newton, 0)
  pltpu.sync_copy(acc_v.at[pl.ds(c * ROWS_W, ROWS_W)],
                  dis_hbm.at[pl.ds(t * ROWS_T + c * ROWS_W, ROWS_W)])

  r0 = t * ROWS_T + c * ROWS_W
  w = t * NC + c
  TAIL = N_NODES - (NW - 1) * ROWS_W

  @pl.when(w < NW - 1)
  def _full_load():
    pltpu.sync_copy(xp_hbm.at[pl.ds(r0, ROWS_W), pl.ds(0, DH)], x0_v)
    pltpu.sync_copy(xp_hbm.at[pl.ds(r0, ROWS_W), pl.ds(DH, DH)], x1_v)

  @pl.when(w == NW - 1)
  def _tail_load():
    pltpu.sync_copy(xp_hbm.at[pl.ds(r0, TAIL), pl.ds(0, DH)],
                    x0_v.at[pl.ds(0, TAIL)])
    pltpu.sync_copy(xp_hbm.at[pl.ds(r0, TAIL), pl.ds(DH, DH)],
                    x1_v.at[pl.ds(0, TAIL)])

  def _scale(r2, _):
    for u in range(2):
      r = r2 * 2 + u
      dv = plsc.load_gather(acc_v,
                            [jnp.zeros((16,), jnp.int32) + (r + c * ROWS_W)])
      for j in range(4):
        x0_v[r, pl.ds(j * 16, 16)] = x0_v[r, pl.ds(j * 16, 16)] * dv
        x1_v[r, pl.ds(j * 16, 16)] = x1_v[r, pl.ds(j * 16, 16)] * dv
    return 0
  lax.fori_loop(0, ROWS_W // 2, _scale, 0)

  @pl.when(w < NW - 1)
  def _full_store():
    pltpu.sync_copy(x0_v, xs0_hbm.at[pl.ds(r0, ROWS_W)])
    pltpu.sync_copy(x1_v, xs1_hbm.at[pl.ds(r0, ROWS_W)])

  @pl.when(w == NW - 1)
  def _tail_store():
    pltpu.sync_copy(x0_v.at[pl.ds(0, TAIL)], xs0_hbm.at[pl.ds(r0, TAIL)])
    pltpu.sync_copy(x1_v.at[pl.ds(0, TAIL)], xs1_hbm.at[pl.ds(r0, TAIL)])


_deg_scale = pl.kernel(
    _deg_scale_body,
    out_type=(jax.ShapeDtypeStruct((NP, DH), jnp.float32),
              jax.ShapeDtypeStruct((NP, DH), jnp.float32),
              jax.ShapeDtypeStruct((NP,), jnp.float32)),
    mesh=_mesh,
    compiler_params=_params,
    scratch_types=(
        pltpu.VMEM((E_TILE,), jnp.int32),
        pltpu.VMEM((NP,), jnp.float32),
        pltpu.VMEM((ROWS_T,), jnp.float32),
        pltpu.VMEM((NS, ROWS_T), jnp.float32),
        pltpu.VMEM((ROWS_W, DH), jnp.float32),
        pltpu.VMEM((ROWS_W, DH), jnp.float32),
        pltpu.VMEM_SHARED((NS, NP), jnp.float32),
    ),
)


def _aggregate_body(xs0_hbm, xs1_hbm, e3d_hbm, aggp_hbm,
                    col_l, row_l, bufs0, bufs1, bufs2, bufs3,
                    gsem0, gsem1, gsem2, gsem3,
                    agg_s):
  c = lax.axis_index("c")
  t = lax.axis_index("s")
  bufs = (bufs0, bufs1, bufs2, bufs3)
  gsems = (gsem0, gsem1, gsem2, gsem3)

  def _zero_buf0():
    def _zb(i, _):
      for j in range(4):
        bufs0[i, pl.ds(j * 16, 16)] = jnp.zeros((16,), jnp.float32)
      return 0
    lax.fori_loop(0, CH, _zb, 0)

  def _zero_own_slice():
    for i in range(5):
      pltpu.sync_copy(bufs0, agg_s.at[pl.ds(t * ROWS_T + i * CH, CH)])

  _zero_buf0()
  _zero_own_slice()

  w = t * NC + c
  base = w * K_W + jnp.minimum(w, NEXTRA)
  pltpu.sync_copy(e3d_hbm.at[1, pl.ds(base, K_W)], col_l.at[pl.ds(0, K_W)])
  pltpu.sync_copy(e3d_hbm.at[0, pl.ds(base, K_W)], row_l.at[pl.ds(0, K_W)])

  @pl.when(w < NEXTRA)
  def _extra_idx():
    pltpu.sync_copy(e3d_hbm.at[1, pl.ds(base + K_W, 1)],
                    col_l.at[pl.ds(K_W, 1)])
    pltpu.sync_copy(e3d_hbm.at[0, pl.ds(base + K_W, 1)],
                    row_l.at[pl.ds(K_W, 1)])

  plsc.subcore_barrier()

  for h, xs_hbm in ((0, xs0_hbm), (1, xs1_hbm)):
    for j in range(NBUF):
      pltpu.async_copy(xs_hbm.at[col_l.at[j]], bufs[j], gsems[j])

    def _ring(i, _):
      for j in range(NBUF):
        k = i * NBUF + j
        pltpu.make_async_copy(xs_hbm.at[col_l.at[k]], bufs[j], gsems[j]).wait()
        pltpu.sync_copy(bufs[j], agg_s.at[row_l.at[k]], add=True)
        pltpu.async_copy(xs_hbm.at[col_l.at[k + NBUF]], bufs[j], gsems[j])
      return 0
    lax.fori_loop(0, (K_W - 6) // NBUF, _ring, 0)

    for j in range(2):
      k = K_W - 6 + j
      pltpu.make_async_copy(xs_hbm.at[col_l.at[k]], bufs[k % NBUF],
                            gsems[k % NBUF]).wait()
      pltpu.sync_copy(bufs[k % NBUF], agg_s.at[row_l.at[k]], add=True)
      pltpu.async_copy(xs_hbm.at[col_l.at[k + NBUF]], bufs[k % NBUF],
                       gsems[k % NBUF])
    for j in range(NBUF):
      k = K_W - NBUF + j
      pltpu.make_async_copy(xs_hbm.at[col_l.at[k]], bufs[k % NBUF],
                            gsems[k % NBUF]).wait()
      pltpu.sync_copy(bufs[k % NBUF], agg_s.at[row_l.at[k]], add=True)

    @pl.when(w < NEXTRA)
    def _extra_chunk():
      pltpu.async_copy(xs_hbm.at[col_l.at[K_W]], bufs0, gsems[0])
      pltpu.make_async_copy(xs_hbm.at[col_l.at[K_W]], bufs0, gsems[0]).wait()
      pltpu.sync_copy(bufs0, agg_s.at[row_l.at[K_W]], add=True)

    plsc.subcore_barrier()

    for i in range(5):
      pltpu.sync_copy(agg_s.at[pl.ds(t * ROWS_T + i * 128, 128)],
                      aggp_hbm.at[c, pl.ds(t * ROWS_T + i * 128, 128),
                                  pl.ds(h * DH, DH)])

    if h == 0:
      _zero_buf0()
      _zero_own_slice()
      plsc.subcore_barrier()


_aggregate = pl.kernel(
    _aggregate_body,
    out_type=jax.ShapeDtypeStruct((NC, NP, D_FEAT), jnp.float32),
    mesh=_mesh,
    compiler_params=_params,
    scratch_types=(
        pltpu.VMEM((K_W + 1, CH), jnp.int32),
        pltpu.VMEM((K_W + 1, CH), jnp.int32),
        pltpu.VMEM((CH, DH), jnp.float32),
        pltpu.VMEM((CH, DH), jnp.float32),
        pltpu.VMEM((CH, DH), jnp.float32),
        pltpu.VMEM((CH, DH), jnp.float32),
        pltpu.SemaphoreType.DMA,
        pltpu.SemaphoreType.DMA,
        pltpu.SemaphoreType.DMA,
        pltpu.SemaphoreType.DMA,
        pltpu.VMEM_SHARED((NP, DH), jnp.float32),
    ),
)


def _project_body(aggp_ref, dis_ref, w_ref, b_ref, out_ref):
  a = (aggp_ref[0] + aggp_ref[1]) * dis_ref[...]
  y = lax.dot_general(a, w_ref[...], (((1,), (1,)), ((), ())),
                      preferred_element_type=jnp.float32)
  out_ref[...] = jnp.maximum(y + b_ref[...], 0.0)


_BM = 1000


def _project(aggp, dis_col, w, b2):
  return pl.pallas_call(
      _project_body,
      grid=(N_NODES // _BM,),
      in_specs=[
          pl.BlockSpec((NC, _BM, D_FEAT), lambda i: (0, i, 0)),
          pl.BlockSpec((_BM, 1), lambda i: (i, 0)),
          pl.BlockSpec((D_FEAT, D_FEAT), lambda i: (0, 0)),
          pl.BlockSpec((1, D_FEAT), lambda i: (0, 0)),
      ],
      out_specs=pl.BlockSpec((_BM, D_FEAT), lambda i: (i, 0)),
      out_shape=jax.ShapeDtypeStruct((N_NODES, D_FEAT), jnp.float32),
  )(aggp, dis_col, w, b2)


@jax.jit
def kernel(x, edge_index, W, b):
  row = edge_index[0]
  e3d = edge_index.reshape(2, KCH, CH)

  xs0, xs1, dis = _deg_scale(row, x)
  aggp = _aggregate(xs0, xs1, e3d)
  dis_col = dis[:N_NODES].reshape(N_NODES, 1)
  return _project(aggp, dis_col, W, b.reshape(1, D_FEAT))


# speedup vs baseline: 1.0186x; 1.0186x over previous
import jax
import jax.numpy as jnp
from jax import lax
from jax.experimental import pallas as pl
from jax.experimental.pallas import tpu as pltpu
from jax.experimental.pallas import tpu_sc as plsc

N_NODES = 10000
N_EDGES = 320000
D_FEAT = 128
DH = D_FEAT // 2

NC = 2
NS = 16
NW = NC * NS

NP = 10240
ROWS_W = NP // NW
ROWS_T = NP // NS

E_TILE = N_EDGES // NS
CH = 128
KCH = N_EDGES // CH
K_W = KCH // NW
NEXTRA = KCH - NW * K_W
NBUF = 4

_mesh = plsc.VectorSubcoreMesh(core_axis_name="c", subcore_axis_name="s")
_params = pltpu.CompilerParams(needs_layout_passes=False,
                               use_tc_tiling_on_sc=False)


def _rsqrt_newton(v):
  i = plsc.bitcast(v, jnp.int32)
  y = plsc.bitcast(jnp.full((16,), 0x5F3759DF, jnp.int32)
                   - lax.shift_right_logical(i, 1), jnp.float32)
  half = v * 0.5
  for _ in range(3):
    y = y * (1.5 - half * y * y)
  return jnp.where(v > 0.5, y, jnp.zeros((16,), jnp.float32))


def _deg_scale_body(row_hbm, xp_hbm, xs0_hbm, xs1_hbm, dis_hbm,
                    row_v, deg_v, acc_v, red_v, x0_v, x1_v,
                    deg_all):
  c = lax.axis_index("c")
  t = lax.axis_index("s")

  def _z(i, _):
    deg_v[pl.ds(i * 16, 16)] = jnp.zeros((16,), jnp.float32)
    return 0
  lax.fori_loop(0, NP // 16, _z, 0)

  pltpu.sync_copy(row_hbm.at[pl.ds(t * E_TILE, E_TILE)], row_v)
  ones = jnp.ones((16,), jnp.float32)

  def _hist(i, _):
    for u in range(5):
      idx = row_v[pl.ds((i * 5 + u) * 16, 16)]
      plsc.addupdate_scatter(deg_v, [idx], ones)
    return 0
  lax.fori_loop(0, E_TILE // 16 // 5, _hist, 0)

  pltpu.sync_copy(deg_v, deg_all.at[t])
  plsc.subcore_barrier()

  pltpu.sync_copy(deg_all.at[pl.ds(0, NS), pl.ds(t * ROWS_T, ROWS_T)], red_v)

  def _add(i, _):
    o = i * 16
    v = red_v[0, pl.ds(o, 16)]
    for j in range(1, NS):
      v = v + red_v[j, pl.ds(o, 16)]
    acc_v[pl.ds(o, 16)] = v
    return 0
  lax.fori_loop(0, ROWS_T // 16, _add, 0)

  def _newton(i, _):
    acc_v[pl.ds(i * 16, 16)] = _rsqrt_newton(acc_v[pl.ds(i * 16, 16)])
    return 0
  lax.fori_loop(0, ROWS_T // 16, _newton, 0)
  pltpu.sync_copy(acc_v.at[pl.ds(c * ROWS_W, ROWS_W)],
                  dis_hbm.at[pl.ds(t * ROWS_T + c * ROWS_W, ROWS_W)])

  r0 = t * ROWS_T + c * ROWS_W
  w = t * NC + c
  TAIL = N_NODES - (NW - 1) * ROWS_W

  @pl.when(w < NW - 1)
  def _full_load():
    pltpu.sync_copy(xp_hbm.at[pl.ds(r0, ROWS_W), pl.ds(0, DH)], x0_v)
    pltpu.sync_copy(xp_hbm.at[pl.ds(r0, ROWS_W), pl.ds(DH, DH)], x1_v)

  @pl.when(w == NW - 1)
  def _tail_load():
    pltpu.sync_copy(xp_hbm.at[pl.ds(r0, TAIL), pl.ds(0, DH)],
                    x0_v.at[pl.ds(0, TAIL)])
    pltpu.sync_copy(xp_hbm.at[pl.ds(r0, TAIL), pl.ds(DH, DH)],
                    x1_v.at[pl.ds(0, TAIL)])

  def _scale(r2, _):
    for u in range(2):
      r = r2 * 2 + u
      dv = plsc.load_gather(acc_v,
                            [jnp.zeros((16,), jnp.int32) + (r + c * ROWS_W)])
      for j in range(4):
        x0_v[r, pl.ds(j * 16, 16)] = x0_v[r, pl.ds(j * 16, 16)] * dv
        x1_v[r, pl.ds(j * 16, 16)] = x1_v[r, pl.ds(j * 16, 16)] * dv
    return 0
  lax.fori_loop(0, ROWS_W // 2, _scale, 0)

  @pl.when(w < NW - 1)
  def _full_store():
    pltpu.sync_copy(x0_v, xs0_hbm.at[pl.ds(r0, ROWS_W)])
    pltpu.sync_copy(x1_v, xs1_hbm.at[pl.ds(r0, ROWS_W)])

  @pl.when(w == NW - 1)
  def _tail_store():
    pltpu.sync_copy(x0_v.at[pl.ds(0, TAIL)], xs0_hbm.at[pl.ds(r0, TAIL)])
    pltpu.sync_copy(x1_v.at[pl.ds(0, TAIL)], xs1_hbm.at[pl.ds(r0, TAIL)])


_deg_scale = pl.kernel(
    _deg_scale_body,
    out_type=(jax.ShapeDtypeStruct((NP, DH), jnp.float32),
              jax.ShapeDtypeStruct((NP, DH), jnp.float32),
              jax.ShapeDtypeStruct((NP,), jnp.float32)),
    mesh=_mesh,
    compiler_params=_params,
    scratch_types=(
        pltpu.VMEM((E_TILE,), jnp.int32),
        pltpu.VMEM((NP,), jnp.float32),
        pltpu.VMEM((ROWS_T,), jnp.float32),
        pltpu.VMEM((NS, ROWS_T), jnp.float32),
        pltpu.VMEM((ROWS_W, DH), jnp.float32),
        pltpu.VMEM((ROWS_W, DH), jnp.float32),
        pltpu.VMEM_SHARED((NS, NP), jnp.float32),
    ),
)


def _aggregate_body(xs0_hbm, xs1_hbm, e3d_hbm, aggp_hbm,
                    col_l, row_l, bufs0, bufs1, bufs2, bufs3,
                    gsem0, gsem1, gsem2, gsem3,
                    agg_s):
  c = lax.axis_index("c")
  t = lax.axis_index("s")
  bufs = (bufs0, bufs1, bufs2, bufs3)
  gsems = (gsem0, gsem1, gsem2, gsem3)

  def _zero_buf0():
    def _zb(i, _):
      for j in range(4):
        bufs0[i, pl.ds(j * 16, 16)] = jnp.zeros((16,), jnp.float32)
      return 0
    lax.fori_loop(0, CH, _zb, 0)

  def _zero_own_slice():
    for i in range(5):
      pltpu.sync_copy(bufs0, agg_s.at[pl.ds(t * ROWS_T + i * CH, CH)])

  _zero_buf0()
  _zero_own_slice()

  w = t * NC + c
  base = w * K_W + jnp.minimum(w, NEXTRA)
  pltpu.sync_copy(e3d_hbm.at[1, pl.ds(base, K_W)], col_l.at[pl.ds(0, K_W)])
  pltpu.sync_copy(e3d_hbm.at[0, pl.ds(base, K_W)], row_l.at[pl.ds(0, K_W)])

  @pl.when(w < NEXTRA)
  def _extra_idx():
    pltpu.sync_copy(e3d_hbm.at[1, pl.ds(base + K_W, 1)],
                    col_l.at[pl.ds(K_W, 1)])
    pltpu.sync_copy(e3d_hbm.at[0, pl.ds(base + K_W, 1)],
                    row_l.at[pl.ds(K_W, 1)])

  plsc.subcore_barrier()

  for h, xs_hbm in ((0, xs0_hbm), (1, xs1_hbm)):
    for j in range(NBUF):
      pltpu.async_copy(xs_hbm.at[col_l.at[j]], bufs[j], gsems[j])

    def _ring(i, _):
      for j in range(NBUF):
        k = i * NBUF + j
        pltpu.make_async_copy(xs_hbm.at[col_l.at[k]], bufs[j], gsems[j]).wait()
        pltpu.sync_copy(bufs[j], agg_s.at[row_l.at[k]], add=True)
        pltpu.async_copy(xs_hbm.at[col_l.at[k + NBUF]], bufs[j], gsems[j])
      return 0
    lax.fori_loop(0, (K_W - 6) // NBUF, _ring, 0)

    for j in range(2):
      k = K_W - 6 + j
      pltpu.make_async_copy(xs_hbm.at[col_l.at[k]], bufs[k % NBUF],
                            gsems[k % NBUF]).wait()
      pltpu.sync_copy(bufs[k % NBUF], agg_s.at[row_l.at[k]], add=True)
      pltpu.async_copy(xs_hbm.at[col_l.at[k + NBUF]], bufs[k % NBUF],
                       gsems[k % NBUF])
    for j in range(NBUF):
      k = K_W - NBUF + j
      pltpu.make_async_copy(xs_hbm.at[col_l.at[k]], bufs[k % NBUF],
                            gsems[k % NBUF]).wait()
      pltpu.sync_copy(bufs[k % NBUF], agg_s.at[row_l.at[k]], add=True)

    @pl.when(w < NEXTRA)
    def _extra_chunk():
      pltpu.async_copy(xs_hbm.at[col_l.at[K_W]], bufs0, gsems[0])
      pltpu.make_async_copy(xs_hbm.at[col_l.at[K_W]], bufs0, gsems[0]).wait()
      pltpu.sync_copy(bufs0, agg_s.at[row_l.at[K_W]], add=True)

    plsc.subcore_barrier()

    for i in range(5):
      pltpu.sync_copy(agg_s.at[pl.ds(t * ROWS_T + i * 128, 128)],
                      aggp_hbm.at[c, pl.ds(t * ROWS_T + i * 128, 128),
                                  pl.ds(h * DH, DH)])

    if h == 0:
      _zero_buf0()
      _zero_own_slice()
      plsc.subcore_barrier()


_aggregate = pl.kernel(
    _aggregate_body,
    out_type=jax.ShapeDtypeStruct((NC, NP, D_FEAT), jnp.float32),
    mesh=_mesh,
    compiler_params=_params,
    scratch_types=(
        pltpu.VMEM((K_W + 1, CH), jnp.int32),
        pltpu.VMEM((K_W + 1, CH), jnp.int32),
        pltpu.VMEM((CH, DH), jnp.float32),
        pltpu.VMEM((CH, DH), jnp.float32),
        pltpu.VMEM((CH, DH), jnp.float32),
        pltpu.VMEM((CH, DH), jnp.float32),
        pltpu.SemaphoreType.DMA,
        pltpu.SemaphoreType.DMA,
        pltpu.SemaphoreType.DMA,
        pltpu.SemaphoreType.DMA,
        pltpu.VMEM_SHARED((NP, DH), jnp.float32),
    ),
)


def _project_body(aggp_ref, dis_ref, w_ref, b_ref, out_ref):
  a = (aggp_ref[0] + aggp_ref[1]) * dis_ref[...]
  y = lax.dot_general(a, w_ref[...], (((1,), (1,)), ((), ())),
                      preferred_element_type=jnp.float32)
  out_ref[...] = jnp.maximum(y + b_ref[...], 0.0)


_BM = 2000


def _project(aggp, dis_col, w, b2):
  return pl.pallas_call(
      _project_body,
      grid=(N_NODES // _BM,),
      in_specs=[
          pl.BlockSpec((NC, _BM, D_FEAT), lambda i: (0, i, 0)),
          pl.BlockSpec((_BM, 1), lambda i: (i, 0)),
          pl.BlockSpec((D_FEAT, D_FEAT), lambda i: (0, 0)),
          pl.BlockSpec((1, D_FEAT), lambda i: (0, 0)),
      ],
      out_specs=pl.BlockSpec((_BM, D_FEAT), lambda i: (i, 0)),
      out_shape=jax.ShapeDtypeStruct((N_NODES, D_FEAT), jnp.float32),
  )(aggp, dis_col, w, b2)


@jax.jit
def kernel(x, edge_index, W, b):
  row = edge_index[0]
  e3d = edge_index.reshape(2, KCH, CH)

  xs0, xs1, dis = _deg_scale(row, x)
  aggp = _aggregate(xs0, xs1, e3d)
  dis_col = dis[:N_NODES].reshape(N_NODES, 1)
  return _project(aggp, dis_col, W, b.reshape(1, D_FEAT))
